# Initial kernel scaffold; baseline (speedup 1.0000x reference)
import jax
import jax.numpy as jnp
from jax.experimental import pallas as pl


def kernel(x, edge_index, edge_attr, message, Wl1, bl1, Wr1, br1, We1, att1, bias1, Wl2, bl2, Wr2, br2, We2, att2, bias2, Wfc, bfc):
    raise NotImplementedError("write your pallas kernel here")


# speedup vs baseline: 14.5406x; 14.5406x over previous
import functools

import jax
import jax.numpy as jnp
from jax import lax
from jax.experimental import pallas as pl
from jax.experimental.pallas import tpu as pltpu
from jax.experimental.pallas import tpu_sc as plsc

N = 10000
E = 640000
EMB = 64
H = 2
D2 = EMB * H
W = 144
B = 80
NBATCH = E // B
NC = 2
NS = 16
NW = NC * NS
BPW = NBATCH // NW
RPS = N // NS
NGRP = B // 16


def _edge_body(xl_hbm, xr_hbm, packed_hbm, params_hbm, zeros_hbm, out_hbm,
               acc, packed_v, xlj, xri, st, params_v, sem0, sem1):
    c = lax.axis_index("c")
    s = lax.axis_index("s")
    wid = s * NC + c
    lane = lax.iota(jnp.int32, 16)

    pltpu.sync_copy(params_hbm, params_v)
    pltpu.sync_copy(zeros_hbm.at[pl.ds(0, B)], st)
    pltpu.sync_copy(zeros_hbm, acc.at[pl.ds(s * RPS, RPS)])
    plsc.subcore_barrier()

    wch = [params_v[0, pl.ds(ch * 16, 16)] for ch in range(D2 // 16)]
    ach = [params_v[1, pl.ds(ch * 16, 16)] for ch in range(D2 // 16)]

    base = wid * BPW

    def batch_body(g, carry):
        row = base + g
        pltpu.sync_copy(packed_hbm.at[row], packed_v)
        cp1 = pltpu.async_copy(xl_hbm.at[packed_v.at[0]], xlj, sem0)
        cp2 = pltpu.async_copy(xr_hbm.at[packed_v.at[1]], xri, sem1)
        cp1.wait()
        cp2.wait()

        def grp_body(grp, carry2):
            eid = grp * 16 + lane
            a = plsc.bitcast(packed_v[2, pl.ds(grp * 16, 16)], jnp.float32)
            acc0 = jnp.zeros((16,), jnp.float32)
            acc1 = jnp.zeros((16,), jnp.float32)
            for k in range(D2):
                kv = jnp.full((16,), k, jnp.int32)
                colL = plsc.load_gather(xlj, [eid, kv])
                colR = plsc.load_gather(xri, [eid, kv])
                z = colL + colR + a * wch[k // 16][k % 16]
                m = jnp.where(z >= 0.0, z, 0.2 * z)
                if k < EMB:
                    acc0 = acc0 + m * ach[k // 16][k % 16]
                else:
                    acc1 = acc1 + m * ach[k // 16][k % 16]
            ea0 = jnp.exp(acc0)
            ea1 = jnp.exp(acc1)
            for k in range(D2):
                kv = jnp.full((16,), k, jnp.int32)
                colL = plsc.load_gather(xlj, [eid, kv])
                plsc.store_scatter(st, [eid, kv],
                                   colL * (ea0 if k < EMB else ea1))
            plsc.store_scatter(st, [eid, jnp.full((16,), D2, jnp.int32)], ea0)
            plsc.store_scatter(st, [eid, jnp.full((16,), D2 + 1, jnp.int32)],
                               ea1)
            return carry2

        lax.fori_loop(0, NGRP, grp_body, 0)

        pltpu.sync_copy(st, acc.at[packed_v.at[1]], add=True)
        return carry

    lax.fori_loop(0, BPW, batch_body, 0)

    plsc.subcore_barrier()
    pltpu.sync_copy(acc.at[pl.ds(s * RPS, RPS)],
                    out_hbm.at[c, pl.ds(s * RPS, RPS)])


_edge_pass = functools.partial(
    pl.kernel,
    out_type=jax.ShapeDtypeStruct((NC, N, W), jnp.float32),
    mesh=plsc.VectorSubcoreMesh(core_axis_name="c", subcore_axis_name="s"),
    compiler_params=pltpu.CompilerParams(use_tc_tiling_on_sc=False,
                                         needs_layout_passes=False),
    scratch_types=[
        pltpu.VMEM_SHARED((N, W), jnp.float32),
        pltpu.VMEM((3, B), jnp.int32),
        pltpu.VMEM((B, D2), jnp.float32),
        pltpu.VMEM((B, D2), jnp.float32),
        pltpu.VMEM((B, W), jnp.float32),
        pltpu.VMEM((2, D2), jnp.float32),
        pltpu.SemaphoreType.DMA,
        pltpu.SemaphoreType.DMA,
    ],
)(_edge_body)


_BLK = 2000


def _prep1_body(x_ref, wl_ref, bl_ref, wr_ref, br_ref, xl_ref, xr_ref):
    xv = x_ref[...]
    xl_ref[...] = xv * wl_ref[...] + bl_ref[...]
    xr_ref[...] = xv * wr_ref[...] + br_ref[...]


def _prep1(x, wl, bl, wr, br):
    return pl.pallas_call(
        _prep1_body,
        grid=(N // _BLK,),
        in_specs=[
            pl.BlockSpec((_BLK, 1), lambda i: (i, 0)),
            pl.BlockSpec((1, D2), lambda i: (0, 0)),
            pl.BlockSpec((1, D2), lambda i: (0, 0)),
            pl.BlockSpec((1, D2), lambda i: (0, 0)),
            pl.BlockSpec((1, D2), lambda i: (0, 0)),
        ],
        out_specs=[
            pl.BlockSpec((_BLK, D2), lambda i: (i, 0)),
            pl.BlockSpec((_BLK, D2), lambda i: (i, 0)),
        ],
        out_shape=[
            jax.ShapeDtypeStruct((N, D2), jnp.float32),
            jax.ShapeDtypeStruct((N, D2), jnp.float32),
        ],
    )(x, wl, bl, wr, br)


def _combine(p, blk):
    pa = p[0]
    pb = p[1]
    un = pa[:, :D2] + pb[:, :D2]
    d0 = pa[:, D2:D2 + 1] + pb[:, D2:D2 + 1]
    d1 = pa[:, D2 + 1:D2 + 2] + pb[:, D2 + 1:D2 + 2]
    den = jnp.concatenate([
        jnp.broadcast_to(d0, (blk, EMB)),
        jnp.broadcast_to(d1, (blk, EMB)),
    ], axis=1)
    return un / (den + 1e-16)


def _mid_body(p_ref, b1_ref, wl_ref, bl_ref, wr_ref, br_ref, xl_ref, xr_ref):
    h1 = jnp.maximum(_combine(p_ref[...], _BLK) + b1_ref[...], 0.0)
    xl_ref[...] = jnp.dot(h1, wl_ref[...],
                          preferred_element_type=jnp.float32) + bl_ref[...]
    xr_ref[...] = jnp.dot(h1, wr_ref[...],
                          preferred_element_type=jnp.float32) + br_ref[...]


def _mid(p, b1, wl, bl, wr, br):
    return pl.pallas_call(
        _mid_body,
        grid=(N // _BLK,),
        in_specs=[
            pl.BlockSpec((NC, _BLK, W), lambda i: (0, i, 0)),
            pl.BlockSpec((1, D2), lambda i: (0, 0)),
            pl.BlockSpec((D2, D2), lambda i: (0, 0)),
            pl.BlockSpec((1, D2), lambda i: (0, 0)),
            pl.BlockSpec((D2, D2), lambda i: (0, 0)),
            pl.BlockSpec((1, D2), lambda i: (0, 0)),
        ],
        out_specs=[
            pl.BlockSpec((_BLK, D2), lambda i: (i, 0)),
            pl.BlockSpec((_BLK, D2), lambda i: (i, 0)),
        ],
        out_shape=[
            jax.ShapeDtypeStruct((N, D2), jnp.float32),
            jax.ShapeDtypeStruct((N, D2), jnp.float32),
        ],
    )(p, b1, wl, bl, wr, br)


def _readout_body(p_ref, b2_ref, msg_ref, wfc_ref, bfc_ref, out_ref):
    h2 = _combine(p_ref[...], N) + b2_ref[...]
    me = jnp.dot(msg_ref[...], wfc_ref[...],
                 preferred_element_type=jnp.float32) + bfc_ref[...]
    dp = jnp.sum(h2 * me, axis=1, keepdims=True)
    mx = jnp.max(dp)
    ex = jnp.exp(dp - mx)
    out_ref[...] = ex / jnp.sum(ex)


def _readout(p, b2, msg, wfc, bfc):
    return pl.pallas_call(
        _readout_body,
        out_shape=jax.ShapeDtypeStruct((N, 1), jnp.float32),
    )(p, b2, msg, wfc, bfc)


def kernel(x, edge_index, edge_attr, message, Wl1, bl1, Wr1, br1, We1, att1,
           bias1, Wl2, bl2, Wr2, br2, We2, att2, bias2, Wfc, bfc):
    src = edge_index[0]
    dst = edge_index[1]
    abits = lax.bitcast_convert_type(edge_attr[:, 0], jnp.int32)
    packed = jnp.stack([
        src.reshape(NBATCH, B),
        dst.reshape(NBATCH, B),
        abits.reshape(NBATCH, B),
    ], axis=1)
    zeros = jnp.zeros((RPS, W), jnp.float32)
    params1 = jnp.stack([We1[0], att1.reshape(D2)])
    params2 = jnp.stack([We2[0], att2.reshape(D2)])

    xl1, xr1 = _prep1(x, Wl1, bl1[None], Wr1, br1[None])
    p1 = _edge_pass(xl1, xr1, packed, params1, zeros)
    xl2, xr2 = _mid(p1, bias1[None], Wl2, bl2[None], Wr2, br2[None])
    p2 = _edge_pass(xl2, xr2, packed, params2, zeros)
    probs = _readout(p2, bias2[None], message, Wfc, bfc[None])
    return probs
